# baseline (speedup 1.0000x reference)
import jax
import jax.numpy as jnp
from jax import lax
from jax.experimental import pallas as pl
from jax.experimental.pallas import tpu as pltpu
from jax.experimental.pallas import tpu_sc as plsc

N_USERS = 1000000
N_ITEMS = 100
N_ITEMS_PAD = 112
N_FACTORS = 64
EMB_DIM = 16
BATCH = 16384

NC = 2
NS = 16
NW = NC * NS
CHUNK = BATCH // NW
NQ = 4
QB = CHUNK // NQ
L = 16
NG = CHUNK // L


def _sc_kernel_body(uidx_hbm, iidx_hbm, uf_hbm, if_hbm, ub_hbm, ib_hbm,
                    me_hbm, wvec_hbm, params_hbm,
                    out_hbm,
                    uidx_v, iidx_v, ubidx_v, umod_v, uf_v, ub_v, if_v, me_v,
                    ib_v, wvec_v, params_v, c_v, out_v, sem):
    wid = lax.axis_index("s") * NC + lax.axis_index("c")

    pltpu.sync_copy(uidx_hbm.at[wid], uidx_v)
    pltpu.sync_copy(iidx_hbm.at[wid], iidx_v)

    for q in range(NQ):
        pltpu.async_copy(uf_hbm.at[uidx_v.at[q]], uf_v.at[pl.ds(q * QB, QB)],
                         sem)

    for q in range(NQ):
        for r in range(QB // L):
            u = uidx_v[q, pl.ds(r * L, L)]
            ubidx_v[q, pl.ds(r * L, L)] = lax.shift_right_logical(u, 4)
            umod_v[pl.ds(q * QB + r * L, L)] = lax.bitwise_and(u, 15)
    for q in range(NQ):
        pltpu.async_copy(ub_hbm.at[ubidx_v.at[q]], ub_v.at[pl.ds(q * QB, QB)],
                         sem)

    pltpu.sync_copy(if_hbm, if_v)
    pltpu.sync_copy(me_hbm, me_v)
    pltpu.sync_copy(ib_hbm, ib_v)
    pltpu.sync_copy(wvec_hbm, wvec_v)
    pltpu.sync_copy(params_hbm, params_v)

    params = params_v[:]
    w_cf = params[0]
    w_cb = params[1]
    b_content = params[2]
    g_bias = params[3]
    wvec = wvec_v[:]
    lanes = lax.iota(jnp.int32, 16)
    zeros_i = jnp.zeros((16,), jnp.int32)

    for t in range(N_ITEMS_PAD // L):
        j_lanes = t * L + lanes
        acc = jnp.zeros((16,), jnp.float32)
        for d in range(EMB_DIM):
            acc = acc + plsc.load_gather(me_v, [j_lanes, zeros_i + d]) * wvec[d]
        ib_lanes = ib_v[pl.ds(t * L, L)]
        c_v[pl.ds(t * L, L)] = w_cb * (acc + b_content) + ib_lanes + g_bias

    for q in range(NQ):
        pltpu.make_async_copy(uf_hbm.at[uidx_v.at[q]],
                              uf_v.at[pl.ds(q * QB, QB)], sem).wait()
    for q in range(NQ):
        pltpu.make_async_copy(ub_hbm.at[ubidx_v.at[q]],
                              ub_v.at[pl.ds(q * QB, QB)], sem).wait()

    def group_body(g, _):
        row_lanes = g * L + lanes
        iidx_lanes = iidx_v[pl.ds(g * L, L)]
        a0 = jnp.zeros((16,), jnp.float32)
        a1 = jnp.zeros((16,), jnp.float32)
        a2 = jnp.zeros((16,), jnp.float32)
        a3 = jnp.zeros((16,), jnp.float32)
        accs = [a0, a1, a2, a3]
        for k in range(N_FACTORS):
            u_k = plsc.load_gather(uf_v, [row_lanes, zeros_i + k])
            v_k = plsc.load_gather(if_v, [iidx_lanes, zeros_i + k])
            accs[k % 4] = accs[k % 4] + u_k * v_k
        dot = (accs[0] + accs[1]) + (accs[2] + accs[3])
        c_lanes = plsc.load_gather(c_v, [iidx_lanes])
        umod_lanes = umod_v[pl.ds(g * L, L)]
        ub_lanes = plsc.load_gather(ub_v, [row_lanes, umod_lanes])
        out_v[pl.ds(g * L, L)] = w_cf * dot + c_lanes + ub_lanes
        return _

    lax.fori_loop(0, NG, group_body, None)

    pltpu.sync_copy(out_v, out_hbm.at[pl.ds(wid * CHUNK, CHUNK)])


@jax.jit
def _run(uidx, iidx, user_factors, item_factors, user_bias,
         item_bias, movie_embeddings, wvec, params):
    mesh = plsc.VectorSubcoreMesh(core_axis_name="c", subcore_axis_name="s")
    f = pl.kernel(
        _sc_kernel_body,
        mesh=mesh,
        compiler_params=pltpu.CompilerParams(needs_layout_passes=False,
                                             use_tc_tiling_on_sc=False),
        out_type=jax.ShapeDtypeStruct((BATCH,), jnp.float32),
        scratch_types=[
            pltpu.VMEM((NQ, QB), jnp.int32),
            pltpu.VMEM((CHUNK,), jnp.int32),
            pltpu.VMEM((NQ, QB), jnp.int32),
            pltpu.VMEM((CHUNK,), jnp.int32),
            pltpu.VMEM((CHUNK, N_FACTORS), jnp.float32),
            pltpu.VMEM((CHUNK, L), jnp.float32),
            pltpu.VMEM((N_ITEMS, N_FACTORS), jnp.float32),
            pltpu.VMEM((N_ITEMS_PAD, EMB_DIM), jnp.float32),
            pltpu.VMEM((N_ITEMS_PAD,), jnp.float32),
            pltpu.VMEM((EMB_DIM,), jnp.float32),
            pltpu.VMEM((16,), jnp.float32),
            pltpu.VMEM((N_ITEMS_PAD + 16,), jnp.float32),
            pltpu.VMEM((CHUNK,), jnp.float32),
            pltpu.SemaphoreType.DMA,
        ],
    )
    return f(uidx, iidx, user_factors, item_factors, user_bias,
             item_bias, movie_embeddings, wvec, params)


def kernel(user_idx, item_idx, user_factors, item_factors, user_bias,
           item_bias, global_bias, movie_embeddings, w_cf, w_cb, W_content,
           b_content):
    uidx = jnp.reshape(user_idx.astype(jnp.int32), (NW, NQ, QB))
    iidx = jnp.reshape(item_idx.astype(jnp.int32), (NW, CHUNK))
    ub = jnp.reshape(user_bias, (N_USERS // L, L))
    ib = jnp.pad(jnp.reshape(item_bias, (N_ITEMS,)),
                 (0, N_ITEMS_PAD - N_ITEMS))
    me = jnp.pad(movie_embeddings, ((0, N_ITEMS_PAD - N_ITEMS), (0, 0)))
    wvec = jnp.reshape(W_content, (EMB_DIM,))
    params = jnp.zeros((16,), jnp.float32)
    params = params.at[0].set(w_cf)
    params = params.at[1].set(w_cb)
    params = params.at[2].set(jnp.reshape(b_content, ())[()])
    params = params.at[3].set(jnp.reshape(global_bias, ())[()])
    return _run(uidx, iidx, user_factors, item_factors, ub, ib,
                me, wvec, params)
